# Initial kernel scaffold; baseline (speedup 1.0000x reference)
import jax
import jax.numpy as jnp
from jax.experimental import pallas as pl


def kernel(x, edge_index, edge_attr, W_in, b_in, W_msg, b_msg, W_upd, b_upd, W_p1, b_p1, W_p2, b_p2, W_node, b_node, W_glob, b_glob):
    raise NotImplementedError("write your pallas kernel here")


# speedup vs baseline: 2.0027x; 2.0027x over previous
import functools
import jax
import jax.numpy as jnp
from jax import lax
from jax.experimental import pallas as pl
from jax.experimental.pallas import tpu as pltpu
from jax.experimental.pallas import tpu_sc as plsc

_N = 50000
_E = 800000
_H = 64
_H2 = 32
_BN = 2000
_BE = 3200
_ER = _E // 128
_NS = 16
_RPT = -(-_ER // _NS)
_BR = 200
_NB = _N // _BR
_BPT = -(-_NB // _NS)


def _in_proj_body(x_ref, win_ref, bin_ref, wa_ref, h_ref, g_ref):
    x = x_ref[...]
    h = x[:, 0:1] * win_ref[0:1, :] + x[:, 1:2] * win_ref[1:2, :] + bin_ref[0:1, :]
    h_ref[...] = h
    g = jnp.dot(h, wa_ref[...], preferred_element_type=jnp.float32)
    g_ref[0] = g[:, :_H2]
    g_ref[1] = g[:, _H2:]


def _edge_c_body(ea_ref, wb_ref, bm_ref, c_ref):
    ea = ea_ref[...]
    c = ea[:, 0:1] * wb_ref[0:1, :] + ea[:, 1:2] * wb_ref[1:2, :] + bm_ref[0:1, :]
    c_ref[0] = c[:, :_H2]
    c_ref[1] = c[:, _H2:]


def _update_body(h_ref, agg_ref, wu_ref, bu_ref, wa_ref, h_out_ref, g_ref, *, with_g):
    h = h_ref[...]
    a = jnp.concatenate([agg_ref[0], agg_ref[1]], axis=1)
    u = (jnp.dot(h, wu_ref[:_H, :], preferred_element_type=jnp.float32)
         + jnp.dot(a, wu_ref[_H:, :], preferred_element_type=jnp.float32)
         + bu_ref[0:1, :])
    hn = jnp.maximum(u, 0.0)
    h_out_ref[...] = hn
    if with_g:
        g = jnp.dot(hn, wa_ref[...], preferred_element_type=jnp.float32)
        g_ref[0] = g[:, :_H2]
        g_ref[1] = g[:, _H2:]


def _heads_body(h_ref, wn_ref, bn_ref, wg_ref, bg_ref, nl_ref, gl_ref, acc_ref):
    i = pl.program_id(0)
    h = h_ref[...]
    nl_ref[...] = jnp.dot(h, wn_ref[...], preferred_element_type=jnp.float32) + bn_ref[0:1, :]

    @pl.when(i == 0)
    def _():
        acc_ref[...] = jnp.zeros_like(acc_ref)

    acc_ref[...] += jnp.sum(h, axis=0, keepdims=True)

    @pl.when(i == pl.num_programs(0) - 1)
    def _():
        hm = acc_ref[...] * (1.0 / _N)
        gl_ref[...] = jnp.dot(hm, wg_ref[...], preferred_element_type=jnp.float32) + bg_ref[0:1, :]


def _full(shape):
    return pl.BlockSpec(shape, lambda i: tuple(0 for _ in shape))


def _in_proj(x, w_in, b_in, a0):
    grid = _N // _BN
    return pl.pallas_call(
        _in_proj_body,
        grid=(grid,),
        in_specs=[
            pl.BlockSpec((_BN, 2), lambda i: (i, 0)),
            _full((2, _H)),
            _full((1, _H)),
            _full((_H, _H)),
        ],
        out_specs=[
            pl.BlockSpec((_BN, _H), lambda i: (i, 0)),
            pl.BlockSpec((2, _BN, _H2), lambda i: (0, i, 0)),
        ],
        out_shape=[
            jax.ShapeDtypeStruct((_N, _H), jnp.float32),
            jax.ShapeDtypeStruct((2, _N, _H2), jnp.float32),
        ],
    )(x, w_in, b_in.reshape(1, _H), a0)


def _edge_c(edge_attr, wb, bm):
    grid = _E // _BE
    return pl.pallas_call(
        _edge_c_body,
        grid=(grid,),
        in_specs=[
            pl.BlockSpec((_BE, 2), lambda i: (i, 0)),
            _full((2, _H)),
            _full((1, _H)),
        ],
        out_specs=pl.BlockSpec((2, _BE, _H2), lambda i: (0, i, 0)),
        out_shape=jax.ShapeDtypeStruct((2, _E, _H2), jnp.float32),
    )(edge_attr, wb, bm.reshape(1, _H))


def _update(h, agg2, wu, bu, wa, with_g):
    grid = _N // _BN
    out_specs = [pl.BlockSpec((_BN, _H), lambda i: (i, 0)),
                 pl.BlockSpec((2, _BN, _H2), lambda i: (0, i, 0))]
    out_shape = [jax.ShapeDtypeStruct((_N, _H), jnp.float32),
                 jax.ShapeDtypeStruct((2, _N, _H2), jnp.float32)]
    res = pl.pallas_call(
        functools.partial(_update_body, with_g=with_g),
        grid=(grid,),
        in_specs=[
            pl.BlockSpec((_BN, _H), lambda i: (i, 0)),
            pl.BlockSpec((2, _BN, _H2), lambda i: (0, i, 0)),
            _full((2 * _H, _H)),
            _full((1, _H)),
            _full((_H, _H)),
        ],
        out_specs=out_specs,
        out_shape=out_shape,
    )(h, agg2, wu, bu.reshape(1, _H), wa)
    return res


def _heads(h, w_node, b_node, w_glob, b_glob):
    grid = _N // _BN
    nl, gl = pl.pallas_call(
        _heads_body,
        grid=(grid,),
        in_specs=[
            pl.BlockSpec((_BN, _H), lambda i: (i, 0)),
            _full((_H, 1)),
            _full((1, 1)),
            _full((_H, 1)),
            _full((1, 1)),
        ],
        out_specs=[
            pl.BlockSpec((_BN, 1), lambda i: (i, 0)),
            pl.BlockSpec((1, 1), lambda i: (0, 0)),
        ],
        out_shape=[
            jax.ShapeDtypeStruct((_N, 1), jnp.float32),
            jax.ShapeDtypeStruct((1, 1), jnp.float32),
        ],
        scratch_shapes=[pltpu.VMEM((1, _H), jnp.float32)],
    )(h, w_node, b_node.reshape(1, 1), w_glob, b_glob.reshape(1, 1))
    return nl, gl


def _sc_edge_body(g_hbm, c_hbm, src_hbm, dst_hbm, out_hbm,
                  table, srcbuf, dstbuf, gbuf, cbuf, zbuf, sem):
    c = lax.axis_index("c")
    s = lax.axis_index("s")

    def _zfill(i, _):
        zbuf[i, pl.ds(0, 16)] = jnp.zeros((16,), jnp.float32)
        zbuf[i, pl.ds(16, 16)] = jnp.zeros((16,), jnp.float32)
        return 0
    lax.fori_loop(0, _BR, _zfill, 0)
    b0 = s * _BPT
    nblk = jnp.minimum(_BPT, _NB - b0)

    def _zcopy(k, _):
        pltpu.sync_copy(zbuf, table.at[pl.ds((b0 + k) * _BR, _BR)])
        return 0
    lax.fori_loop(0, nblk, _zcopy, 0)
    plsc.subcore_barrier()

    row0 = s * _RPT
    nrows = jnp.minimum(_RPT, _ER - row0)
    goff = c * _N
    coff = c * _E

    def _edge_row(k, _):
        row = row0 + k
        pltpu.sync_copy(src_hbm.at[pl.ds(row * 128, 128)], srcbuf)
        pltpu.sync_copy(dst_hbm.at[pl.ds(row * 128, 128)], dstbuf)

        def _off(i, _):
            srcbuf[pl.ds(i * 16, 16)] = srcbuf[pl.ds(i * 16, 16)] + goff
            return 0
        lax.fori_loop(0, 8, _off, 0)
        gcp = pltpu.async_copy(g_hbm.at[srcbuf], gbuf, sem)
        pltpu.sync_copy(c_hbm.at[pl.ds(coff + row * 128, 128)], cbuf)
        gcp.wait()

        def _relu_row(r, _):
            v0 = gbuf[r, pl.ds(0, 16)] + cbuf[r, pl.ds(0, 16)]
            gbuf[r, pl.ds(0, 16)] = jnp.maximum(v0, 0.0)
            v1 = gbuf[r, pl.ds(16, 16)] + cbuf[r, pl.ds(16, 16)]
            gbuf[r, pl.ds(16, 16)] = jnp.maximum(v1, 0.0)
            return 0
        lax.fori_loop(0, 128, _relu_row, 0)
        pltpu.sync_copy(gbuf, table.at[dstbuf], add=True)
        return 0
    lax.fori_loop(0, nrows, _edge_row, 0)
    plsc.subcore_barrier()

    def _wcopy(k, _):
        pltpu.sync_copy(table.at[pl.ds((b0 + k) * _BR, _BR)],
                        out_hbm.at[pl.ds(c * _N + (b0 + k) * _BR, _BR)])
        return 0
    lax.fori_loop(0, nblk, _wcopy, 0)


@functools.lru_cache(maxsize=None)
def _sc_edge_kernel():
    return pl.kernel(
        _sc_edge_body,
        out_type=jax.ShapeDtypeStruct((2 * _N, _H2), jnp.float32),
        mesh=plsc.VectorSubcoreMesh(core_axis_name="c", subcore_axis_name="s",
                                    num_cores=2, num_subcores=_NS),
        compiler_params=pltpu.CompilerParams(use_tc_tiling_on_sc=False),
        scratch_types=[
            pltpu.VMEM_SHARED((_N, _H2), jnp.float32),
            pltpu.VMEM((128,), jnp.int32),
            pltpu.VMEM((128,), jnp.int32),
            pltpu.VMEM((128, _H2), jnp.float32),
            pltpu.VMEM((128, _H2), jnp.float32),
            pltpu.VMEM((_BR, _H2), jnp.float32),
            pltpu.SemaphoreType.DMA,
        ],
    )


def kernel(x, edge_index, edge_attr, W_in, b_in, W_msg, b_msg, W_upd, b_upd,
           W_p1, b_p1, W_p2, b_p2, W_node, b_node, W_glob, b_glob):
    src1d = edge_index[0]
    dst1d = edge_index[1]

    h, g2 = _in_proj(x, W_in, b_in, W_msg[0, :_H, :])
    for l in range(3):
        c2 = _edge_c(edge_attr, W_msg[l, _H:, :], b_msg[l])
        gflat = g2.reshape(2 * _N, _H2)
        cflat = c2.reshape(2 * _E, _H2)
        aggflat = _sc_edge_kernel()(gflat, cflat, src1d, dst1d)
        agg2 = aggflat.reshape(2, _N, _H2)
        wa = W_msg[l + 1, :_H, :] if l < 2 else W_msg[0, :_H, :]
        h, g2 = _update(h, agg2, W_upd[l], b_upd[l], wa, with_g=(l < 2))
    nl, gl = _heads(h, W_node, b_node, W_glob, b_glob)
    return nl.reshape(_N), gl.reshape(1)


# speedup vs baseline: 2.4987x; 1.2477x over previous
import functools
import jax
import jax.numpy as jnp
from jax import lax
from jax.experimental import pallas as pl
from jax.experimental.pallas import tpu as pltpu
from jax.experimental.pallas import tpu_sc as plsc

_N = 50000
_E = 800000
_H = 64
_H2 = 32
_BN = 2000
_BE = 3200
_ER = _E // 128
_NS = 16
_RPT = -(-_ER // _NS)
_BR = 80
_NB = _N // _BR
_BPT = -(-_NB // _NS)


def _rbf(v):
    return v.astype(jnp.bfloat16).astype(jnp.float32)


def _in_proj_body(x_ref, win_ref, bin_ref, wa_ref, h_ref, g_ref):
    x = _rbf(x_ref[...])
    w = _rbf(win_ref[...])
    h = x[:, 0:1] * w[0:1, :] + x[:, 1:2] * w[1:2, :] + bin_ref[0:1, :]
    h_ref[...] = h
    g = jnp.dot(_rbf(h), _rbf(wa_ref[...]), preferred_element_type=jnp.float32, precision=lax.Precision.HIGHEST)
    g_ref[0] = g[:, :_H2]
    g_ref[1] = g[:, _H2:]


def _edge_c_body(ea_ref, wb_ref, bm_ref, c_ref):
    ea = _rbf(ea_ref[...])
    wb = _rbf(wb_ref[...])
    c = ea[:, 0:1] * wb[0:1, :] + ea[:, 1:2] * wb[1:2, :] + bm_ref[0:1, :]
    c_ref[0] = c[:, :_H2]
    c_ref[1] = c[:, _H2:]


def _update_body(h_ref, agg_ref, wu_ref, bu_ref, wa_ref, h_out_ref, g_ref, *, with_g):
    h = h_ref[...]
    a = jnp.concatenate([agg_ref[0], agg_ref[1]], axis=1)
    wu = _rbf(wu_ref[...])
    u = (jnp.dot(_rbf(h), wu[:_H, :], preferred_element_type=jnp.float32, precision=lax.Precision.HIGHEST)
         + jnp.dot(_rbf(a), wu[_H:, :], preferred_element_type=jnp.float32, precision=lax.Precision.HIGHEST)
         + bu_ref[0:1, :])
    hn = jnp.maximum(u, 0.0)
    h_out_ref[...] = hn
    if with_g:
        g = jnp.dot(_rbf(hn), _rbf(wa_ref[...]), preferred_element_type=jnp.float32, precision=lax.Precision.HIGHEST)
        g_ref[0] = g[:, :_H2]
        g_ref[1] = g[:, _H2:]


def _heads_body(h_ref, wn_ref, bn_ref, wg_ref, bg_ref, nl_ref, gl_ref, acc_ref):
    i = pl.program_id(0)
    h = h_ref[...]
    nl_ref[...] = jnp.dot(_rbf(h), _rbf(wn_ref[...]), preferred_element_type=jnp.float32, precision=lax.Precision.HIGHEST) + bn_ref[0:1, :]

    @pl.when(i == 0)
    def _():
        acc_ref[...] = jnp.zeros_like(acc_ref)

    acc_ref[...] += jnp.sum(h, axis=0, keepdims=True)

    @pl.when(i == pl.num_programs(0) - 1)
    def _():
        hm = acc_ref[...] * (1.0 / _N)
        gl_ref[...] = jnp.dot(_rbf(hm), _rbf(wg_ref[...]), preferred_element_type=jnp.float32, precision=lax.Precision.HIGHEST) + bg_ref[0:1, :]


def _full(shape):
    return pl.BlockSpec(shape, lambda i: tuple(0 for _ in shape))


def _in_proj(x, w_in, b_in, a0):
    grid = _N // _BN
    return pl.pallas_call(
        _in_proj_body,
        grid=(grid,),
        in_specs=[
            pl.BlockSpec((_BN, 2), lambda i: (i, 0)),
            _full((2, _H)),
            _full((1, _H)),
            _full((_H, _H)),
        ],
        out_specs=[
            pl.BlockSpec((_BN, _H), lambda i: (i, 0)),
            pl.BlockSpec((2, _BN, _H2), lambda i: (0, i, 0)),
        ],
        out_shape=[
            jax.ShapeDtypeStruct((_N, _H), jnp.float32),
            jax.ShapeDtypeStruct((2, _N, _H2), jnp.float32),
        ],
    )(x, w_in, b_in.reshape(1, _H), a0)


def _edge_c(edge_attr, wb, bm):
    grid = _E // _BE
    return pl.pallas_call(
        _edge_c_body,
        grid=(grid,),
        in_specs=[
            pl.BlockSpec((_BE, 2), lambda i: (i, 0)),
            _full((2, _H)),
            _full((1, _H)),
        ],
        out_specs=pl.BlockSpec((2, _BE, _H2), lambda i: (0, i, 0)),
        out_shape=jax.ShapeDtypeStruct((2, _E, _H2), jnp.float32),
    )(edge_attr, wb, bm.reshape(1, _H))


def _update(h, agg2, wu, bu, wa, with_g):
    grid = _N // _BN
    out_specs = [pl.BlockSpec((_BN, _H), lambda i: (i, 0)),
                 pl.BlockSpec((2, _BN, _H2), lambda i: (0, i, 0))]
    out_shape = [jax.ShapeDtypeStruct((_N, _H), jnp.float32),
                 jax.ShapeDtypeStruct((2, _N, _H2), jnp.float32)]
    res = pl.pallas_call(
        functools.partial(_update_body, with_g=with_g),
        grid=(grid,),
        in_specs=[
            pl.BlockSpec((_BN, _H), lambda i: (i, 0)),
            pl.BlockSpec((2, _BN, _H2), lambda i: (0, i, 0)),
            _full((2 * _H, _H)),
            _full((1, _H)),
            _full((_H, _H)),
        ],
        out_specs=out_specs,
        out_shape=out_shape,
    )(h, agg2, wu, bu.reshape(1, _H), wa)
    return res


def _heads(h, w_node, b_node, w_glob, b_glob):
    grid = _N // _BN
    nl, gl = pl.pallas_call(
        _heads_body,
        grid=(grid,),
        in_specs=[
            pl.BlockSpec((_BN, _H), lambda i: (i, 0)),
            _full((_H, 1)),
            _full((1, 1)),
            _full((_H, 1)),
            _full((1, 1)),
        ],
        out_specs=[
            pl.BlockSpec((_BN, 1), lambda i: (i, 0)),
            pl.BlockSpec((1, 1), lambda i: (0, 0)),
        ],
        out_shape=[
            jax.ShapeDtypeStruct((_N, 1), jnp.float32),
            jax.ShapeDtypeStruct((1, 1), jnp.float32),
        ],
        scratch_shapes=[pltpu.VMEM((1, _H), jnp.float32)],
    )(h, w_node, b_node.reshape(1, 1), w_glob, b_glob.reshape(1, 1))
    return nl, gl


def _sc_edge_body(g_hbm, c_hbm, src_hbm, dst_hbm, out_hbm,
                  table,
                  src0, src1, src2, dst0, dst1, dst2,
                  gb0, gb1, gb2, mb0, mb1, mb2,
                  zbuf, sem,
                  sio0, sio1, sio2, sg0, sg1, sg2, ss0, ss1, ss2):
    c = lax.axis_index("c")
    s = lax.axis_index("s")
    srcb = (src0, src1, src2)
    dstb = (dst0, dst1, dst2)
    gb = (gb0, gb1, gb2)
    mb = (mb0, mb1, mb2)
    sio = (sio0, sio1, sio2)
    sg = (sg0, sg1, sg2)
    ss = (ss0, ss1, ss2)

    def _zfill(i, _):
        zbuf[i, pl.ds(0, 16)] = jnp.zeros((16,), jnp.float32)
        zbuf[i, pl.ds(16, 16)] = jnp.zeros((16,), jnp.float32)
        return 0
    lax.fori_loop(0, _BR, _zfill, 0)
    b0 = s * _BPT
    nblk = jnp.minimum(_BPT, _NB - b0)

    def _zcopy(k, _):
        pltpu.sync_copy(zbuf, table.at[pl.ds((b0 + k) * _BR, _BR)])
        return 0
    lax.fori_loop(0, nblk, _zcopy, 0)
    plsc.subcore_barrier()

    row0 = s * _RPT
    nrows = jnp.minimum(_RPT, _ER - row0)
    goff = c * _N
    coff = c * _E

    def _issue_io(row, b):
        pltpu.async_copy(src_hbm.at[pl.ds(row * 128, 128)], srcb[b], sio[b])
        pltpu.async_copy(dst_hbm.at[pl.ds(row * 128, 128)], dstb[b], sio[b])
        pltpu.async_copy(c_hbm.at[pl.ds(coff + row * 128, 128)], gb[b], sio[b])

    def _wait_io(row, b):
        pltpu.make_async_copy(src_hbm.at[pl.ds(row * 128, 128)], srcb[b], sio[b]).wait()
        pltpu.make_async_copy(dst_hbm.at[pl.ds(row * 128, 128)], dstb[b], sio[b]).wait()
        pltpu.make_async_copy(c_hbm.at[pl.ds(coff + row * 128, 128)], gb[b], sio[b]).wait()

    def _wait_scatter(b):
        pltpu.make_async_copy(gb[b], table.at[dstb[b]], ss[b]).wait()

    def _stage(t, b):
        ra = t
        rb = t - 1
        rc = t - 2
        sa = b
        sb_ = (b + 2) % 3
        sc_ = (b + 1) % 3

        @pl.when(ra < nrows)
        def _():
            @pl.when(ra >= 3)
            def _():
                _wait_scatter(sa)
            _issue_io(row0 + ra, sa)

        @pl.when((rb >= 0) & (rb < nrows))
        def _():
            _wait_io(row0 + rb, sb_)

            def _off(i, _):
                srcb[sb_][pl.ds(i * 16, 16)] = srcb[sb_][pl.ds(i * 16, 16)] + goff
                return 0
            lax.fori_loop(0, 8, _off, 0)
            pltpu.async_copy(g_hbm.at[srcb[sb_]], mb[sb_], sg[sb_])

        @pl.when((rc >= 0) & (rc < nrows))
        def _():
            pltpu.make_async_copy(g_hbm.at[srcb[sc_]], mb[sc_], sg[sc_]).wait()
            gbb = gb[sc_]
            mbb = mb[sc_]

            def _rr(rr, _):
                for u in range(8):
                    r = rr * 8 + u
                    gbb[r, pl.ds(0, 16)] = jnp.maximum(
                        gbb[r, pl.ds(0, 16)] + mbb[r, pl.ds(0, 16)], 0.0)
                    gbb[r, pl.ds(16, 16)] = jnp.maximum(
                        gbb[r, pl.ds(16, 16)] + mbb[r, pl.ds(16, 16)], 0.0)
                return 0
            lax.fori_loop(0, 16, _rr, 0)
            pltpu.async_copy(gbb, table.at[dstb[sc_]], ss[sc_], add=True)

    def _triple(it, _):
        for b in range(3):
            _stage(it * 3 + b, b)
        return 0
    lax.fori_loop(0, (_RPT + 2 + 2) // 3, _triple, 0)
    for b in range(3):
        _wait_scatter(b)
    plsc.subcore_barrier()

    def _wcopy(k, _):
        pltpu.sync_copy(table.at[pl.ds((b0 + k) * _BR, _BR)],
                        out_hbm.at[pl.ds(c * _N + (b0 + k) * _BR, _BR)])
        return 0
    lax.fori_loop(0, nblk, _wcopy, 0)


@functools.lru_cache(maxsize=None)
def _sc_edge_kernel():
    return pl.kernel(
        _sc_edge_body,
        out_type=jax.ShapeDtypeStruct((2 * _N, _H2), jnp.float32),
        mesh=plsc.VectorSubcoreMesh(core_axis_name="c", subcore_axis_name="s",
                                    num_cores=2, num_subcores=_NS),
        compiler_params=pltpu.CompilerParams(use_tc_tiling_on_sc=False),
        scratch_types=(
            [pltpu.VMEM_SHARED((_N, _H2), jnp.float32)]
            + [pltpu.VMEM((128,), jnp.int32) for _ in range(6)]
            + [pltpu.VMEM((128, _H2), jnp.float32) for _ in range(6)]
            + [pltpu.VMEM((_BR, _H2), jnp.float32)]
            + [pltpu.SemaphoreType.DMA for _ in range(10)]
        ),
    )


def kernel(x, edge_index, edge_attr, W_in, b_in, W_msg, b_msg, W_upd, b_upd,
           W_p1, b_p1, W_p2, b_p2, W_node, b_node, W_glob, b_glob):
    src1d = edge_index[0]
    dst1d = edge_index[1]

    h, g2 = _in_proj(x, W_in, b_in, W_msg[0, :_H, :])
    for l in range(3):
        c2 = _edge_c(edge_attr, W_msg[l, _H:, :], b_msg[l])
        gflat = g2.reshape(2 * _N, _H2)
        cflat = c2.reshape(2 * _E, _H2)
        aggflat = _sc_edge_kernel()(gflat, cflat, src1d, dst1d)
        agg2 = aggflat.reshape(2, _N, _H2)
        wa = W_msg[l + 1, :_H, :] if l < 2 else W_msg[0, :_H, :]
        h, g2 = _update(h, agg2, W_upd[l], b_upd[l], wa, with_g=(l < 2))
    nl, gl = _heads(h, W_node, b_node, W_glob, b_glob)
    return nl.reshape(_N), gl.reshape(1)
